# baseline (speedup 1.0000x reference)
import functools

import jax
import jax.numpy as jnp
from jax.experimental import pallas as pl
from jax.experimental.pallas import tpu as pltpu


def _fused_kernel(x_ref, ws_ref, w2_ref, o_ref, w2s_ref, acc_ref, lines_ref,
                  *, w, c):
    p = pl.program_id(0)
    j = pl.program_id(1)
    nb, _, h, _ = x_ref.shape

    @pl.when(p == 0)
    def _():
        @pl.when(j == 0)
        def _():
            acc_ref[...] = jnp.zeros_like(acc_ref)

        v = jnp.max(jnp.abs(x_ref[...]), axis=0)
        r_mid = jnp.max(v[:, 1:h - 1, :], axis=1)
        p0 = jnp.maximum(r_mid, v[:, 0, :])
        p2 = jnp.maximum(r_mid, v[:, h - 1, :])
        p1 = jnp.maximum(p0, v[:, h - 1, :])

        cols = []
        for pp in (p0, p1, p2):
            cfirst = pp[:, 0:1]
            clast = pp[:, w - 1:w]
            cmid = jnp.max(pp[:, 1:w - 1], axis=1, keepdims=True)
            cols.append(jnp.maximum(cfirst, cmid))
            cols.append(jnp.maximum(jnp.maximum(cfirst, cmid), clast))
            cols.append(jnp.maximum(cmid, clast))
        s = jnp.concatenate(cols, axis=1)
        acc_ref[...] = jnp.maximum(acc_ref[...], s)

    @pl.when((p == 1) & (j == 0))
    def _():
        acc = acc_ref[...]
        act = jnp.concatenate(
            [jnp.transpose(acc[:, i:i + 1]) for i in range(9)],
            axis=1)
        den = jnp.sqrt(ws_ref[...])
        scale = jnp.sqrt(act) / jnp.where(den == 0.0, 1.0, den)
        scale = jnp.where(den == 0.0, 1.0, scale)
        scale = jnp.where(scale == 0.0, 1.0, scale)
        w2s_ref[...] = w2_ref[...] * scale
        inv = 1.0 / scale

        kk = 3 * c
        tr0 = inv[:, 0:kk] + inv[:, kk:2 * kk]
        tr1 = tr0 + inv[:, 2 * kk:3 * kk]
        tr2 = inv[:, kk:2 * kk] + inv[:, 2 * kk:3 * kk]

        def line(tr):
            e0 = tr[:, 0:c] + tr[:, c:2 * c]
            e1 = e0 + tr[:, 2 * c:3 * c]
            e2 = tr[:, c:2 * c] + tr[:, 2 * c:3 * c]
            return jnp.concatenate([e0, jnp.broadcast_to(e1, (w - 2, c)),
                                    e2], axis=0)

        lines_ref[0] = line(tr0)
        lines_ref[1] = line(tr1)
        lines_ref[2] = line(tr2)

    @pl.when(p == 1)
    def _():
        a = jnp.transpose(x_ref[...], (0, 2, 1, 3))
        t = jnp.transpose(a, (0, 1, 3, 2))
        o_ref[...] = t * lines_ref[1][None, None]
        o_ref[:, 0] = t[:, 0] * lines_ref[0][None]
        o_ref[:, h - 1] = t[:, h - 1] * lines_ref[2][None]


def kernel(x, weight):
    b, c, h, w = x.shape
    o = weight.shape[0]
    k = weight.shape[2]
    sf = 0.5
    kkc = k * k * c

    x = x.astype(jnp.float32)
    weight = weight.astype(jnp.float32)

    del sf
    w2 = jnp.transpose(weight, (0, 2, 3, 1)).reshape(o, kkc)
    ws = jnp.max(jnp.abs(w2), axis=0)[None]

    nb = 4 if b % 8 == 0 else (2 if b % 4 == 0 else 1)
    nj = b // nb

    x_out, w2s = pl.pallas_call(
        functools.partial(_fused_kernel, w=w, c=c),
        out_shape=(jax.ShapeDtypeStruct((b, h, w, c), jnp.float32),
                   jax.ShapeDtypeStruct((o, kkc), jnp.float32)),
        grid=(2, nj),
        in_specs=[pl.BlockSpec((nb, c, h, w), lambda p, j: (j, 0, 0, 0)),
                  pl.BlockSpec((1, kkc), lambda p, j: (0, 0)),
                  pl.BlockSpec((o, kkc), lambda p, j: (0, 0))],
        out_specs=(pl.BlockSpec((nb, h, w, c), lambda p, j: (p * j, 0, 0, 0)),
                   pl.BlockSpec((o, kkc), lambda p, j: (0, 0))),
        scratch_shapes=[pltpu.VMEM((c, 9), jnp.float32),
                        pltpu.VMEM((3, w, c), jnp.float32)],
        compiler_params=pltpu.CompilerParams(
            dimension_semantics=("arbitrary", "arbitrary")),
    )(x, ws, w2)

    w_out = w2s.reshape(o, k, k, c)
    return w_out, x_out


# speedup vs baseline: 1.0336x; 1.0336x over previous
import functools

import jax
import jax.numpy as jnp
from jax.experimental import pallas as pl
from jax.experimental.pallas import tpu as pltpu


def _fused_kernel(x_ref, ws_ref, o_ref, amax_ref, acc_ref, lines_ref, *, w, c):
    p = pl.program_id(0)
    j = pl.program_id(1)
    nj = pl.num_programs(1)
    nb, _, h, _ = x_ref.shape

    @pl.when(p == 0)
    def _():
        @pl.when(j == 0)
        def _():
            acc_ref[...] = jnp.zeros_like(acc_ref)

        v = jnp.max(jnp.abs(x_ref[...]), axis=0)
        r_mid = jnp.max(v[:, 1:h - 1, :], axis=1)
        p0 = jnp.maximum(r_mid, v[:, 0, :])
        p2 = jnp.maximum(r_mid, v[:, h - 1, :])
        p1 = jnp.maximum(p0, v[:, h - 1, :])

        cols = []
        for pp in (p0, p1, p2):
            cfirst = pp[:, 0:1]
            clast = pp[:, w - 1:w]
            cmid = jnp.max(pp[:, 1:w - 1], axis=1, keepdims=True)
            cols.append(jnp.maximum(cfirst, cmid))
            cols.append(jnp.maximum(jnp.maximum(cfirst, cmid), clast))
            cols.append(jnp.maximum(cmid, clast))
        s = jnp.concatenate(cols, axis=1)
        acc_ref[...] = jnp.maximum(acc_ref[...], s)

        @pl.when(j == nj - 1)
        def _():
            amax_ref[...] = acc_ref[...]

    @pl.when((p == 1) & (j == 0))
    def _():
        act = acc_ref[...]
        den = jnp.sqrt(ws_ref[...])
        scale = jnp.sqrt(act) / jnp.where(den == 0.0, 1.0, den)
        scale = jnp.where(den == 0.0, 1.0, scale)
        scale = jnp.where(scale == 0.0, 1.0, scale)
        inv = 1.0 / scale

        tr0 = inv[:, 0:3] + inv[:, 3:6]
        tr1 = tr0 + inv[:, 6:9]
        tr2 = inv[:, 3:6] + inv[:, 6:9]

        def line(tr):
            e0 = jnp.transpose(tr[:, 0:1] + tr[:, 1:2])
            e1 = jnp.transpose(tr[:, 0:1] + tr[:, 1:2] + tr[:, 2:3])
            e2 = jnp.transpose(tr[:, 1:2] + tr[:, 2:3])
            return jnp.concatenate([e0, jnp.broadcast_to(e1, (w - 2, c)),
                                    e2], axis=0)

        lines_ref[0] = line(tr0)
        lines_ref[1] = line(tr1)
        lines_ref[2] = line(tr2)

    @pl.when(p == 1)
    def _():
        a = jnp.transpose(x_ref[...], (0, 2, 1, 3))
        t = jnp.transpose(a, (0, 1, 3, 2))
        o_ref[...] = t * lines_ref[1][None, None]
        o_ref[:, 0] = t[:, 0] * lines_ref[0][None]
        o_ref[:, h - 1] = t[:, h - 1] * lines_ref[2][None]


def kernel(x, weight):
    b, c, h, w = x.shape
    o = weight.shape[0]
    k = weight.shape[2]
    sf = 0.5
    kkc = k * k * c

    x = x.astype(jnp.float32)
    weight = weight.astype(jnp.float32)

    w2 = jnp.transpose(weight, (0, 2, 3, 1)).reshape(o, kkc)
    ws9 = jnp.max(jnp.abs(w2), axis=0).reshape(k * k, c)
    ws_c9 = jnp.transpose(ws9)

    nb = 4 if b % 8 == 0 else (2 if b % 4 == 0 else 1)
    nj = b // nb

    x_out, amax = pl.pallas_call(
        functools.partial(_fused_kernel, w=w, c=c),
        out_shape=(jax.ShapeDtypeStruct((b, h, w, c), jnp.float32),
                   jax.ShapeDtypeStruct((c, 9), jnp.float32)),
        grid=(2, nj),
        in_specs=[pl.BlockSpec((nb, c, h, w), lambda p, j: (j, 0, 0, 0)),
                  pl.BlockSpec((c, 9), lambda p, j: (0, 0))],
        out_specs=(pl.BlockSpec((nb, h, w, c), lambda p, j: (p * j, 0, 0, 0)),
                   pl.BlockSpec((c, 9), lambda p, j: (0, 0))),
        scratch_shapes=[pltpu.VMEM((c, 9), jnp.float32),
                        pltpu.VMEM((3, w, c), jnp.float32)],
        compiler_params=pltpu.CompilerParams(
            dimension_semantics=("arbitrary", "arbitrary")),
    )(x, ws_c9)

    act9 = jnp.transpose(amax)
    den = ws9 ** (1.0 - sf)
    scale = (act9 ** sf) / jnp.where(den == 0.0, 1.0, den)
    scale = jnp.where(den == 0.0, 1.0, scale)
    scale = jnp.where(scale == 0.0, 1.0, scale)
    w_out = (w2 * scale.reshape(kkc)).reshape(o, k, k, c)

    return w_out, x_out
